# Initial kernel scaffold; baseline (speedup 1.0000x reference)
import jax
import jax.numpy as jnp
from jax.experimental import pallas as pl


def kernel(users, items, user_emb, item_emb, adj_rows, adj_cols, adj_vals):
    raise NotImplementedError("write your pallas kernel here")


# speedup vs baseline: 5.2273x; 5.2273x over previous
import functools

import jax
import jax.numpy as jnp
from jax import lax
from jax.experimental import pallas as pl
from jax.experimental.pallas import tpu as pltpu
from jax.experimental.pallas import tpu_sc as plsc

_USER = 5000
_N = 10000
_NPAD = 10240
_EMB = 128
_E = 320000
_BATCH = 4096
_NC, _NS = 2, 16
_NW = _NC * _NS
_K = 80
_EPT = _E // _NW
_CH = _EPT // _K
_NG = 5
_G = _CH // _NG
_RPT = _NPAD // _NS
_ZCH = _RPT // _K

_mesh = plsc.VectorSubcoreMesh(core_axis_name="c", subcore_axis_name="s")


def _bvec(v16, j):
    return jnp.take(v16, jnp.full((16,), j, dtype=jnp.int32))


@functools.partial(
    pl.kernel,
    out_type=jax.ShapeDtypeStruct((_NC, _NPAD, _EMB), jnp.float32),
    mesh=_mesh,
    scratch_types=dict(
        colbuf=pltpu.VMEM((_G, _K), jnp.int32),
        rowbuf=pltpu.VMEM((_G, _K), jnp.int32),
        valbuf=pltpu.VMEM((_G, _K), jnp.float32),
        gbuf=pltpu.VMEM((_K, _EMB), jnp.float32),
        sem=pltpu.SemaphoreType.DMA,
        acc=pltpu.VMEM_SHARED((_NPAD, _EMB), jnp.float32),
    ),
)
def _layer(ego, colsr, rowsr, valsr, part, *, colbuf, rowbuf, valbuf, gbuf,
           sem, acc):
    cid = lax.axis_index("c")
    sid = lax.axis_index("s")
    wid = cid * _NS + sid

    z16 = jnp.zeros((16,), jnp.float32)

    def _zero_row(i, carry):
        for cc in range(_EMB // 16):
            gbuf[i, pl.ds(cc * 16, 16)] = z16
        return carry

    lax.fori_loop(0, _K, _zero_row, 0)
    for z in range(_ZCH):
        pltpu.sync_copy(gbuf, acc.at[pl.ds(sid * _RPT + z * _K, _K)])

    plsc.subcore_barrier()

    def _group(grp, carry):
        pltpu.sync_copy(colsr.at[wid, grp], colbuf)
        pltpu.sync_copy(rowsr.at[wid, grp], rowbuf)
        pltpu.sync_copy(valsr.at[wid, grp], valbuf)

        def _chunk(t, c2):
            pltpu.async_copy(ego.at[colbuf.at[t]], gbuf, sem).wait()
            for g in range(_K // 16):
                v16 = valbuf[t, pl.ds(g * 16, 16)]
                for j in range(16):
                    e = g * 16 + j
                    bv = _bvec(v16, j)
                    for cc in range(_EMB // 16):
                        gbuf[e, pl.ds(cc * 16, 16)] = (
                            gbuf[e, pl.ds(cc * 16, 16)] * bv)
            pltpu.sync_copy(gbuf, acc.at[rowbuf.at[t]], add=True)
            return c2

        lax.fori_loop(0, _G, _chunk, 0)
        return carry

    lax.fori_loop(0, _NG, _group, 0)
    plsc.subcore_barrier()

    rstart = sid * _RPT
    pltpu.sync_copy(acc.at[pl.ds(rstart, _RPT)],
                    part.at[cid, pl.ds(rstart, _RPT)])


_CBLK = 512


def _combine_body(p_ref, o_ref):
    o_ref[...] = p_ref[0] + p_ref[1]


def _combine(part):
    return pl.pallas_call(
        _combine_body,
        out_shape=jax.ShapeDtypeStruct((_NPAD, _EMB), jnp.float32),
        grid=(_NPAD // _CBLK,),
        in_specs=[pl.BlockSpec((_NC, _CBLK, _EMB), lambda i: (0, i, 0))],
        out_specs=pl.BlockSpec((_CBLK, _EMB), lambda i: (i, 0)),
    )(part)


_B2 = 2 * _BATCH
_GK = 32
_GCH = _B2 // _GK // _NW


@functools.partial(
    pl.kernel,
    out_type=jax.ShapeDtypeStruct((_B2, _EMB), jnp.float32),
    mesh=_mesh,
    scratch_types=dict(
        ibuf=pltpu.VMEM((_GCH, _GK), jnp.int32),
        gbuf=pltpu.VMEM((_GK, _EMB), jnp.float32),
        obuf=pltpu.VMEM((_GK, _EMB), jnp.float32),
        sem=pltpu.SemaphoreType.DMA,
    ),
)
def _gather_mean(ego0, ego1, ego2, p3a, p3b, bidx, out, *, ibuf, gbuf, obuf,
                 sem):
    cid = lax.axis_index("c")
    sid = lax.axis_index("s")
    wid = cid * _NS + sid
    pltpu.sync_copy(bidx.at[wid], ibuf)
    quarter = jnp.full((16,), 0.25, dtype=jnp.float32)

    def _chunk(t, carry):
        def _acc_rows(first):
            def _row(i, c2):
                for cc in range(_EMB // 16):
                    g = gbuf[i, pl.ds(cc * 16, 16)]
                    if first:
                        obuf[i, pl.ds(cc * 16, 16)] = g
                    else:
                        obuf[i, pl.ds(cc * 16, 16)] = (
                            obuf[i, pl.ds(cc * 16, 16)] + g)
                return c2
            lax.fori_loop(0, _GK, _row, 0)

        for s, src in enumerate((ego0, ego1, ego2, p3a, p3b)):
            pltpu.async_copy(src.at[ibuf.at[t]], gbuf, sem).wait()
            _acc_rows(s == 0)

        def _scale(i, c2):
            for cc in range(_EMB // 16):
                obuf[i, pl.ds(cc * 16, 16)] = (
                    obuf[i, pl.ds(cc * 16, 16)] * quarter)
            return c2

        lax.fori_loop(0, _GK, _scale, 0)
        pltpu.sync_copy(obuf, out.at[pl.ds((wid * _GCH + t) * _GK, _GK)])
        return carry

    lax.fori_loop(0, _GCH, _chunk, 0)


def kernel(users, items, user_emb, item_emb, adj_rows, adj_cols, adj_vals):
    ego0 = jnp.concatenate([user_emb, item_emb], axis=0)
    colsr = adj_cols.reshape(_NW, _NG, _G, _K)
    rowsr = adj_rows.reshape(_NW, _NG, _G, _K)
    valsr = adj_vals.reshape(_NW, _NG, _G, _K)

    p1 = _layer(ego0, colsr, rowsr, valsr)
    ego1 = _combine(p1)
    p2 = _layer(ego1, colsr, rowsr, valsr)
    ego2 = _combine(p2)
    p3 = _layer(ego2, colsr, rowsr, valsr)

    bidx = jnp.concatenate([users, items + _USER]).reshape(_NW, _GCH, _GK)
    out = _gather_mean(ego0, ego1, ego2, p3[0], p3[1], bidx)
    return out[:_BATCH], out[_BATCH:]
